# Initial kernel scaffold; baseline (speedup 1.0000x reference)
import jax
import jax.numpy as jnp
from jax.experimental import pallas as pl


def kernel(hidden_states, gate_w, W1, b1, W2, b2, W3, b3, W4, b4):
    raise NotImplementedError("write your pallas kernel here")


# speedup vs baseline: 1.2804x; 1.2804x over previous
import jax
import jax.numpy as jnp
from jax.experimental import pallas as pl
from jax.experimental.pallas import tpu as pltpu

E = 8
TOP_K = 2
D_IN = 2048
D_H = 1024
D_OUT = 1024


def _elu(h):
    return jnp.where(h > 0, h, jnp.exp(jnp.minimum(h, 0.0)) - 1.0)


def _router_kernel(x_ref, gwt_ref, i1_ref, i2_ref, w1_ref, w2_ref):
    x = x_ref[...]
    gwt = gwt_ref[...]
    logits = jnp.dot(x, gwt, preferred_element_type=jnp.float32)
    m = jnp.max(logits, axis=1, keepdims=True)
    p = jnp.exp(logits - m)
    probs = p / jnp.sum(p, axis=1, keepdims=True)
    iota = jax.lax.broadcasted_iota(jnp.int32, probs.shape, 1)
    v1 = jnp.max(probs, axis=1, keepdims=True)
    i1 = jnp.min(jnp.where(probs >= v1, iota, E), axis=1, keepdims=True)
    probs2 = jnp.where(iota == i1, -1.0, probs)
    v2 = jnp.max(probs2, axis=1, keepdims=True)
    i2 = jnp.min(jnp.where(probs2 >= v2, iota, E), axis=1, keepdims=True)
    s = v1 + v2
    i1_ref[...] = i1
    i2_ref[...] = i2
    w1_ref[...] = v1 / s
    w2_ref[...] = v2 / s


def _moe_dense_kernel(i1_ref, i2_ref, w1_ref, w2_ref, x_ref,
                      W1_ref, W2_ref, W3_ref, W4_ref,
                      b1_ref, b2_ref, b3_ref, b4_ref, out_ref):
    e = pl.program_id(0)
    x = x_ref[...]
    h = jnp.dot(x, W1_ref[0], preferred_element_type=jnp.float32) + b1_ref[0]
    h = _elu(h).astype(jnp.bfloat16)
    h = jnp.dot(h, W2_ref[0], preferred_element_type=jnp.float32) + b2_ref[0]
    h = _elu(h).astype(jnp.bfloat16)
    h = jnp.dot(h, W3_ref[0], preferred_element_type=jnp.float32) + b3_ref[0]
    h = _elu(h).astype(jnp.bfloat16)
    y = jnp.dot(h, W4_ref[0], preferred_element_type=jnp.float32) + b4_ref[0]
    wcol = (jnp.where(i1_ref[...] == e, w1_ref[...], 0.0)
            + jnp.where(i2_ref[...] == e, w2_ref[...], 0.0))
    contrib = wcol * y

    @pl.when(e == 0)
    def _():
        out_ref[...] = contrib

    @pl.when(e != 0)
    def _():
        out_ref[...] += contrib


def kernel(hidden_states, gate_w, W1, b1, W2, b2, W3, b3, W4, b4):
    bsz, seq, d = hidden_states.shape
    S = bsz * seq
    xf = hidden_states.reshape(S, d)
    gwt = gate_w.T

    i1, i2, w1, w2 = pl.pallas_call(
        _router_kernel,
        out_shape=[
            jax.ShapeDtypeStruct((S, 1), jnp.int32),
            jax.ShapeDtypeStruct((S, 1), jnp.int32),
            jax.ShapeDtypeStruct((S, 1), jnp.float32),
            jax.ShapeDtypeStruct((S, 1), jnp.float32),
        ],
    )(xf, gwt)

    x_bf = xf.astype(jnp.bfloat16)
    W1b = W1.astype(jnp.bfloat16)
    W2b = W2.astype(jnp.bfloat16)
    W3b = W3.astype(jnp.bfloat16)
    W4b = W4.astype(jnp.bfloat16)
    b1r = b1.reshape(E, 1, D_H)
    b2r = b2.reshape(E, 1, D_H)
    b3r = b3.reshape(E, 1, D_H)
    b4r = b4.reshape(E, 1, D_OUT)

    full2 = lambda a, b: pl.BlockSpec((a, b), lambda e: (0, 0))
    per_e3 = lambda a, b: pl.BlockSpec((1, a, b), lambda e: (e, 0, 0))

    out = pl.pallas_call(
        _moe_dense_kernel,
        grid=(E,),
        in_specs=[
            full2(S, 1), full2(S, 1), full2(S, 1), full2(S, 1),
            full2(S, D_IN),
            per_e3(D_IN, D_H), per_e3(D_H, D_H), per_e3(D_H, D_H),
            per_e3(D_H, D_OUT),
            per_e3(1, D_H), per_e3(1, D_H), per_e3(1, D_H), per_e3(1, D_OUT),
        ],
        out_specs=pl.BlockSpec((S, D_OUT), lambda e: (0, 0)),
        out_shape=jax.ShapeDtypeStruct((S, D_OUT), jnp.float32),
        compiler_params=pltpu.CompilerParams(
            dimension_semantics=("arbitrary",),
        ),
    )(i1, i2, w1, w2, x_bf, W1b, W2b, W3b, W4b, b1r, b2r, b3r, b4r)

    return out.reshape(bsz, seq, D_OUT)
